# baseline (speedup 1.0000x reference)
import jax
import jax.numpy as jnp
import numpy as np
from jax import lax
from jax.experimental import pallas as pl
from jax.experimental.pallas import tpu as pltpu
from jax.experimental.pallas import tpu_sc as plsc

_NMS_THR = 0.4
_K = 1000
_MAX_OUT = 300
_R, _C = 160, 128
_P = _R * _C
_KPAD = 1024
_TRASH = 1020
_NEG = np.float32(-1e30)
_BIGI = np.int32(2 ** 22)


def _score_kernel(cls_ref, scores_ref, pos_ref):
    scores = jnp.max(cls_ref[...], axis=2)
    idx = (lax.broadcasted_iota(jnp.int32, (_R, _C), 0) * _C
           + lax.broadcasted_iota(jnp.int32, (_R, _C), 1))

    def bs_val(_, carry):
        lo, hi = carry
        mid = (lo + hi) // 2
        t = lax.bitcast_convert_type(mid, jnp.float32)
        cnt = jnp.sum((scores >= t).astype(jnp.int32))
        big = cnt >= _K
        return jnp.where(big, mid, lo), jnp.where(big, hi, mid)

    lo, hi = lax.fori_loop(0, 31, bs_val, (jnp.int32(0), jnp.int32(0x3F800000)))
    vk = lax.bitcast_convert_type(lo, jnp.float32)

    cnt_gt = jnp.sum((scores > vk).astype(jnp.int32))
    need = _K - cnt_gt
    eq = scores == vk

    def bs_idx(_, carry):
        lo2, hi2 = carry
        mid = (lo2 + hi2) // 2
        cnt = jnp.sum((eq & (idx <= mid)).astype(jnp.int32))
        ok = cnt >= need
        return jnp.where(ok, lo2, mid), jnp.where(ok, mid, hi2)

    lo2, hi2 = lax.fori_loop(0, 16, bs_idx, (jnp.int32(-1), jnp.int32(_P - 1)))
    keep = (scores > vk) | (eq & (idx <= hi2))

    kf = keep.astype(jnp.float32)
    ut = (lax.broadcasted_iota(jnp.int32, (_C, _C), 0)
          <= lax.broadcasted_iota(jnp.int32, (_C, _C), 1)).astype(jnp.float32)
    cs = jnp.dot(kf, ut, preferred_element_type=jnp.float32)
    row_tot = cs[:, _C - 1:_C]
    tl = (lax.broadcasted_iota(jnp.int32, (_R, _R), 1)
          < lax.broadcasted_iota(jnp.int32, (_R, _R), 0)).astype(jnp.float32)
    row_off = jnp.dot(tl, row_tot, preferred_element_type=jnp.float32)
    rank = (row_off + cs - kf).astype(jnp.int32)

    scores_ref[...] = scores
    pos_ref[...] = jnp.where(keep, rank, _TRASH)


def _sc_body(pos_hbm, ar_hbm, x1_hbm, y1_hbm, x2_hbm, y2_hbm, sc_hbm,
             oidx_hbm, f0_hbm, f1_hbm, f2_hbm, f3_hbm, f4_hbm,
             pos_v, ar_v, oi_sp, oi_v, f_sp, fb_v, sem):
    wid = lax.axis_index("s") * 2 + lax.axis_index("c")

    @pl.when(wid == 0)
    def _():
        pltpu.sync_copy(pos_hbm, pos_v)
        pltpu.sync_copy(ar_hbm, ar_v)

        pltpu.async_copy(ar_v, oi_sp.at[pos_v], sem).wait()

        pltpu.sync_copy(oi_sp, oi_v)
        pltpu.sync_copy(oi_v, oidx_hbm)
        for j in range(_KPAD // 16):
            v = oi_v[pl.ds(j * 16, 16)]
            oi_v[pl.ds(j * 16, 16)] = jnp.clip(v, 0, _P - 1)

        for fin, fout in ((x1_hbm, f0_hbm), (y1_hbm, f1_hbm),
                          (x2_hbm, f2_hbm), (y2_hbm, f3_hbm),
                          (sc_hbm, f4_hbm)):
            pltpu.sync_copy(fin, f_sp)
            pltpu.async_copy(f_sp.at[oi_v], fb_v, sem).wait()
            pltpu.sync_copy(fb_v, fout)


def _sc_compact(pos2, ar2, x1, y1, x2, y2, sc):
    mesh = plsc.VectorSubcoreMesh(core_axis_name="c", subcore_axis_name="s")
    f = pl.kernel(
        _sc_body,
        out_type=(
            jax.ShapeDtypeStruct((_KPAD,), jnp.int32),
            jax.ShapeDtypeStruct((_KPAD,), jnp.float32),
            jax.ShapeDtypeStruct((_KPAD,), jnp.float32),
            jax.ShapeDtypeStruct((_KPAD,), jnp.float32),
            jax.ShapeDtypeStruct((_KPAD,), jnp.float32),
            jax.ShapeDtypeStruct((_KPAD,), jnp.float32),
        ),
        mesh=mesh,
        scratch_types=[
            pltpu.VMEM((_P,), jnp.int32),
            pltpu.VMEM((_P,), jnp.int32),
            pltpu.VMEM_SHARED((_KPAD,), jnp.int32),
            pltpu.VMEM((_KPAD,), jnp.int32),
            pltpu.VMEM_SHARED((_P,), jnp.float32),
            pltpu.VMEM((_KPAD,), jnp.float32),
            pltpu.SemaphoreType.DMA,
        ],
    )
    return f(pos2, ar2, x1, y1, x2, y2, sc)


def _nms_kernel(x1_ref, y1_ref, x2_ref, y2_ref, sc_ref, ci_ref, det_ref,
                out_ref):
    slot = (lax.broadcasted_iota(jnp.int32, (8, _C), 0) * _C
            + lax.broadcasted_iota(jnp.int32, (8, _C), 1))
    real = slot < _K
    x1 = jnp.where(real, x1_ref[...], 0.0)
    y1 = jnp.where(real, y1_ref[...], 0.0)
    x2 = jnp.where(real, x2_ref[...], 0.0)
    y2 = jnp.where(real, y2_ref[...], 0.0)
    cidx = jnp.where(real, ci_ref[...], _BIGI)
    sw0 = jnp.where(real, sc_ref[...], _NEG)
    area_b = jnp.maximum(x2 - x1, 0.0) * jnp.maximum(y2 - y1, 0.0)
    neg_half = _NEG / 2

    def nms_body(i, sw):
        m = jnp.max(sw)
        is_valid = m > neg_half
        bidx = jnp.min(jnp.where(sw == m, cidx, _BIGI * 2))
        bsel = cidx == bidx
        bx1 = jnp.sum(jnp.where(bsel, x1, 0.0))
        by1 = jnp.sum(jnp.where(bsel, y1, 0.0))
        bx2 = jnp.sum(jnp.where(bsel, x2, 0.0))
        by2 = jnp.sum(jnp.where(bsel, y2, 0.0))
        ix1 = jnp.maximum(bx1, x1)
        iy1 = jnp.maximum(by1, y1)
        ix2 = jnp.minimum(bx2, x2)
        iy2 = jnp.minimum(by2, y2)
        inter = jnp.maximum(ix2 - ix1, 0.0) * jnp.maximum(iy2 - iy1, 0.0)
        area_a = jnp.maximum(bx2 - bx1, 0.0) * jnp.maximum(by2 - by1, 0.0)
        union = area_a + area_b - inter
        iou = inter / jnp.maximum(union, 1e-9)
        suppress = (iou > _NMS_THR) | bsel
        sw = jnp.where(is_valid & suppress, _NEG, sw)
        bi = jnp.minimum(bidx, det_ref.shape[0] - 1)
        row = det_ref[pl.ds(bi, 1), :]
        out_ref[pl.ds(i, 1), :] = jnp.where(is_valid, row, 0.0)
        return sw

    lax.fori_loop(0, _MAX_OUT, nms_body, sw0)


def kernel(boxes, classification, detections):
    b = boxes[0]
    cls = classification[0]
    det = detections[0]
    n = b.shape[0]
    pad = _P - n
    clsp = jnp.pad(cls, ((0, pad), (0, 0)), constant_values=-1e30)
    cls_t = clsp.reshape(_R, _C, cls.shape[1])

    scores, pos2 = pl.pallas_call(
        _score_kernel,
        out_shape=(
            jax.ShapeDtypeStruct((_R, _C), jnp.float32),
            jax.ShapeDtypeStruct((_R, _C), jnp.int32),
        ),
    )(cls_t)

    bp = jnp.pad(b, ((0, pad), (0, 0)))
    ar2 = jnp.arange(_P, dtype=jnp.int32)
    oidx, f0, f1, f2, f3, f4 = _sc_compact(
        pos2.reshape(_P), ar2, bp[:, 0], bp[:, 1], bp[:, 2], bp[:, 3],
        scores.reshape(_P))

    out = pl.pallas_call(
        _nms_kernel,
        out_shape=jax.ShapeDtypeStruct((_MAX_OUT, det.shape[1]), jnp.float32),
    )(f0.reshape(8, _C), f1.reshape(8, _C), f2.reshape(8, _C),
      f3.reshape(8, _C), f4.reshape(8, _C), oidx.reshape(8, _C), det)
    return out[None]


# speedup vs baseline: 1.7363x; 1.7363x over previous
import jax
import jax.numpy as jnp
import numpy as np
from jax import lax
from jax.experimental import pallas as pl
from jax.experimental.pallas import tpu as pltpu
from jax.experimental.pallas import tpu_sc as plsc

_NMS_THR = 0.4
_K = 1000
_MAX_OUT = 300
_R, _C = 160, 128
_P = _R * _C
_KPAD = 1024
_TRASH = 1020
_NEG = np.float32(-1e30)
_BIGI = np.int32(2 ** 22)


def _score_kernel(cls_ref, scores_ref, pos_ref):
    scores = jnp.max(cls_ref[...], axis=0)
    idx = (lax.broadcasted_iota(jnp.int32, (_R, _C), 0) * _C
           + lax.broadcasted_iota(jnp.int32, (_R, _C), 1))

    def bs_val(_, carry):
        lo, hi = carry
        mid = (lo + hi) // 2
        t = lax.bitcast_convert_type(mid, jnp.float32)
        cnt = jnp.sum((scores >= t).astype(jnp.int32))
        big = cnt >= _K
        return jnp.where(big, mid, lo), jnp.where(big, hi, mid)

    lo, hi = lax.fori_loop(0, 31, bs_val, (jnp.int32(0), jnp.int32(0x3F800000)))
    vk = lax.bitcast_convert_type(lo, jnp.float32)

    cnt_gt = jnp.sum((scores > vk).astype(jnp.int32))
    need = _K - cnt_gt
    eq = scores == vk

    def bs_idx(_, carry):
        lo2, hi2 = carry
        mid = (lo2 + hi2) // 2
        cnt = jnp.sum((eq & (idx <= mid)).astype(jnp.int32))
        ok = cnt >= need
        return jnp.where(ok, lo2, mid), jnp.where(ok, mid, hi2)

    lo2, hi2 = lax.fori_loop(0, 16, bs_idx, (jnp.int32(-1), jnp.int32(_P - 1)))
    keep = (scores > vk) | (eq & (idx <= hi2))

    kf = keep.astype(jnp.float32)
    ut = (lax.broadcasted_iota(jnp.int32, (_C, _C), 0)
          <= lax.broadcasted_iota(jnp.int32, (_C, _C), 1)).astype(jnp.float32)
    cs = jnp.dot(kf, ut, preferred_element_type=jnp.float32)
    row_tot = cs[:, _C - 1:_C]
    tl = (lax.broadcasted_iota(jnp.int32, (_R, _R), 1)
          < lax.broadcasted_iota(jnp.int32, (_R, _R), 0)).astype(jnp.float32)
    row_off = jnp.dot(tl, row_tot, preferred_element_type=jnp.float32)
    rank = (row_off + cs - kf).astype(jnp.int32)

    scores_ref[...] = scores
    pos_ref[...] = jnp.where(keep, rank, _TRASH)


def _sc_body(pos_hbm, ar_hbm, x1_hbm, y1_hbm, x2_hbm, y2_hbm, sc_hbm,
             oidx_hbm, f0_hbm, f1_hbm, f2_hbm, f3_hbm, f4_hbm,
             pos_v, ar_v, oi_sp, oi_v, f_sp, fb_v, sem):
    wid = lax.axis_index("s") * 2 + lax.axis_index("c")

    @pl.when(wid == 0)
    def _():
        pltpu.sync_copy(pos_hbm, pos_v)
        pltpu.sync_copy(ar_hbm, ar_v)

        pltpu.async_copy(ar_v, oi_sp.at[pos_v], sem).wait()

        pltpu.sync_copy(oi_sp, oi_v)
        pltpu.sync_copy(oi_v, oidx_hbm)
        for j in range(_KPAD // 16):
            v = oi_v[pl.ds(j * 16, 16)]
            oi_v[pl.ds(j * 16, 16)] = jnp.clip(v, 0, _P - 1)

        for fin, fout in ((x1_hbm, f0_hbm), (y1_hbm, f1_hbm),
                          (x2_hbm, f2_hbm), (y2_hbm, f3_hbm),
                          (sc_hbm, f4_hbm)):
            pltpu.sync_copy(fin, f_sp)
            pltpu.async_copy(f_sp.at[oi_v], fb_v, sem).wait()
            pltpu.sync_copy(fb_v, fout)


def _sc_compact(pos2, ar2, x1, y1, x2, y2, sc):
    mesh = plsc.VectorSubcoreMesh(core_axis_name="c", subcore_axis_name="s")
    f = pl.kernel(
        _sc_body,
        out_type=(
            jax.ShapeDtypeStruct((_KPAD,), jnp.int32),
            jax.ShapeDtypeStruct((_KPAD,), jnp.float32),
            jax.ShapeDtypeStruct((_KPAD,), jnp.float32),
            jax.ShapeDtypeStruct((_KPAD,), jnp.float32),
            jax.ShapeDtypeStruct((_KPAD,), jnp.float32),
            jax.ShapeDtypeStruct((_KPAD,), jnp.float32),
        ),
        mesh=mesh,
        scratch_types=[
            pltpu.VMEM((_P,), jnp.int32),
            pltpu.VMEM((_P,), jnp.int32),
            pltpu.VMEM_SHARED((_KPAD,), jnp.int32),
            pltpu.VMEM((_KPAD,), jnp.int32),
            pltpu.VMEM_SHARED((_P,), jnp.float32),
            pltpu.VMEM((_KPAD,), jnp.float32),
            pltpu.SemaphoreType.DMA,
        ],
    )
    return f(pos2, ar2, x1, y1, x2, y2, sc)


def _nms_kernel(x1_ref, y1_ref, x2_ref, y2_ref, sc_ref, ci_ref, det_ref,
                out_ref):
    slot = (lax.broadcasted_iota(jnp.int32, (8, _C), 0) * _C
            + lax.broadcasted_iota(jnp.int32, (8, _C), 1))
    real = slot < _K
    x1 = jnp.where(real, x1_ref[...], 0.0)
    y1 = jnp.where(real, y1_ref[...], 0.0)
    x2 = jnp.where(real, x2_ref[...], 0.0)
    y2 = jnp.where(real, y2_ref[...], 0.0)
    cidx = jnp.where(real, ci_ref[...], _BIGI)
    sw0 = jnp.where(real, sc_ref[...], _NEG)
    area_b = jnp.maximum(x2 - x1, 0.0) * jnp.maximum(y2 - y1, 0.0)
    neg_half = _NEG / 2

    def nms_body(i, sw):
        m = jnp.max(sw)
        is_valid = m > neg_half
        bidx = jnp.min(jnp.where(sw == m, cidx, _BIGI * 2))
        bsel = cidx == bidx
        bx1 = jnp.sum(jnp.where(bsel, x1, 0.0))
        by1 = jnp.sum(jnp.where(bsel, y1, 0.0))
        bx2 = jnp.sum(jnp.where(bsel, x2, 0.0))
        by2 = jnp.sum(jnp.where(bsel, y2, 0.0))
        ix1 = jnp.maximum(bx1, x1)
        iy1 = jnp.maximum(by1, y1)
        ix2 = jnp.minimum(bx2, x2)
        iy2 = jnp.minimum(by2, y2)
        inter = jnp.maximum(ix2 - ix1, 0.0) * jnp.maximum(iy2 - iy1, 0.0)
        area_a = jnp.maximum(bx2 - bx1, 0.0) * jnp.maximum(by2 - by1, 0.0)
        union = area_a + area_b - inter
        iou = inter / jnp.maximum(union, 1e-9)
        suppress = (iou > _NMS_THR) | bsel
        sw = jnp.where(is_valid & suppress, _NEG, sw)
        bi = jnp.minimum(bidx, det_ref.shape[0] - 1)
        row = det_ref[pl.ds(bi, 1), :]
        out_ref[pl.ds(i, 1), :] = jnp.where(is_valid, row, 0.0)
        return sw

    lax.fori_loop(0, _MAX_OUT, nms_body, sw0)


def kernel(boxes, classification, detections):
    b = boxes[0]
    cls = classification[0]
    det = detections[0]
    n = b.shape[0]
    pad = _P - n
    clsp = jnp.pad(cls, ((0, pad), (0, 0)), constant_values=-1e30)
    cls_t = clsp.T.reshape(cls.shape[1], _R, _C)

    scores, pos2 = pl.pallas_call(
        _score_kernel,
        out_shape=(
            jax.ShapeDtypeStruct((_R, _C), jnp.float32),
            jax.ShapeDtypeStruct((_R, _C), jnp.int32),
        ),
    )(cls_t)

    bp = jnp.pad(b, ((0, pad), (0, 0)))
    ar2 = jnp.arange(_P, dtype=jnp.int32)
    oidx, f0, f1, f2, f3, f4 = _sc_compact(
        pos2.reshape(_P), ar2, bp[:, 0], bp[:, 1], bp[:, 2], bp[:, 3],
        scores.reshape(_P))

    out = pl.pallas_call(
        _nms_kernel,
        out_shape=jax.ShapeDtypeStruct((_MAX_OUT, det.shape[1]), jnp.float32),
    )(f0.reshape(8, _C), f1.reshape(8, _C), f2.reshape(8, _C),
      f3.reshape(8, _C), f4.reshape(8, _C), oidx.reshape(8, _C), det)
    return out[None]


# speedup vs baseline: 1.8517x; 1.0664x over previous
import jax
import jax.numpy as jnp
import numpy as np
from jax import lax
from jax.experimental import pallas as pl
from jax.experimental.pallas import tpu as pltpu
from jax.experimental.pallas import tpu_sc as plsc

_NMS_THR = 0.4
_K = 1000
_MAX_OUT = 300
_R, _C = 160, 128
_P = _R * _C
_KPAD = 1024
_TRASH = 1020
_NEG = np.float32(-1e30)
_BIGI = np.int32(2 ** 22)


def _score_kernel(cls_ref, scores_ref, pos_ref):
    scores = jnp.max(cls_ref[...], axis=0)
    idx = (lax.broadcasted_iota(jnp.int32, (_R, _C), 0) * _C
           + lax.broadcasted_iota(jnp.int32, (_R, _C), 1))

    def bs_val(_, carry):
        lo, hi = carry
        mid = (lo + hi) // 2
        t = lax.bitcast_convert_type(mid, jnp.float32)
        cnt = jnp.sum((scores >= t).astype(jnp.int32))
        big = cnt >= _K
        return jnp.where(big, mid, lo), jnp.where(big, hi, mid)

    lo, hi = lax.fori_loop(0, 31, bs_val, (jnp.int32(0), jnp.int32(0x3F800000)))
    vk = lax.bitcast_convert_type(lo, jnp.float32)

    cnt_gt = jnp.sum((scores > vk).astype(jnp.int32))
    need = _K - cnt_gt
    eq = scores == vk

    def bs_idx(_, carry):
        lo2, hi2 = carry
        mid = (lo2 + hi2) // 2
        cnt = jnp.sum((eq & (idx <= mid)).astype(jnp.int32))
        ok = cnt >= need
        return jnp.where(ok, lo2, mid), jnp.where(ok, mid, hi2)

    lo2, hi2 = lax.fori_loop(0, 16, bs_idx, (jnp.int32(-1), jnp.int32(_P - 1)))
    keep = (scores > vk) | (eq & (idx <= hi2))

    kf = keep.astype(jnp.float32)
    ut = (lax.broadcasted_iota(jnp.int32, (_C, _C), 0)
          <= lax.broadcasted_iota(jnp.int32, (_C, _C), 1)).astype(jnp.float32)
    cs = jnp.dot(kf, ut, preferred_element_type=jnp.float32)
    row_tot = cs[:, _C - 1:_C]
    tl = (lax.broadcasted_iota(jnp.int32, (_R, _R), 1)
          < lax.broadcasted_iota(jnp.int32, (_R, _R), 0)).astype(jnp.float32)
    row_off = jnp.dot(tl, row_tot, preferred_element_type=jnp.float32)
    rank = (row_off + cs - kf).astype(jnp.int32)

    scores_ref[...] = scores
    pos_ref[...] = jnp.where(keep, rank, _TRASH)


def _sc_body(pos_hbm, ar_hbm, x1_hbm, y1_hbm, x2_hbm, y2_hbm, sc_hbm,
             oidx_hbm, f0_hbm, f1_hbm, f2_hbm, f3_hbm, f4_hbm,
             pos_v, ar_v, oi_sp, oi_v, f_sp, fb_v, sem):
    wid = lax.axis_index("s") * 2 + lax.axis_index("c")

    @pl.when(wid == 0)
    def _():
        pltpu.sync_copy(pos_hbm, pos_v)
        pltpu.sync_copy(ar_hbm, ar_v)

        pltpu.async_copy(ar_v, oi_sp.at[pos_v], sem).wait()

        pltpu.sync_copy(oi_sp, oi_v)
        for j in range(_KPAD // 16):
            v = oi_v[pl.ds(j * 16, 16)]
            oi_v[pl.ds(j * 16, 16)] = jnp.clip(v, 0, _P - 1)
        pltpu.sync_copy(oi_v, oidx_hbm)

        for fin, fout in ((x1_hbm, f0_hbm), (y1_hbm, f1_hbm),
                          (x2_hbm, f2_hbm), (y2_hbm, f3_hbm),
                          (sc_hbm, f4_hbm)):
            pltpu.sync_copy(fin, f_sp)
            pltpu.async_copy(f_sp.at[oi_v], fb_v, sem).wait()
            pltpu.sync_copy(fb_v, fout)


def _sc_compact(pos2, ar2, x1, y1, x2, y2, sc):
    mesh = plsc.VectorSubcoreMesh(core_axis_name="c", subcore_axis_name="s")
    f = pl.kernel(
        _sc_body,
        out_type=(
            jax.ShapeDtypeStruct((_KPAD,), jnp.int32),
            jax.ShapeDtypeStruct((_KPAD,), jnp.float32),
            jax.ShapeDtypeStruct((_KPAD,), jnp.float32),
            jax.ShapeDtypeStruct((_KPAD,), jnp.float32),
            jax.ShapeDtypeStruct((_KPAD,), jnp.float32),
            jax.ShapeDtypeStruct((_KPAD,), jnp.float32),
        ),
        mesh=mesh,
        scratch_types=[
            pltpu.VMEM((_P,), jnp.int32),
            pltpu.VMEM((_P,), jnp.int32),
            pltpu.VMEM_SHARED((_KPAD,), jnp.int32),
            pltpu.VMEM((_KPAD,), jnp.int32),
            pltpu.VMEM_SHARED((_P,), jnp.float32),
            pltpu.VMEM((_KPAD,), jnp.float32),
            pltpu.SemaphoreType.DMA,
        ],
    )
    return f(pos2, ar2, x1, y1, x2, y2, sc)


def _nms_kernel(x1_ref, y1_ref, x2_ref, y2_ref, sc_ref,
                x1s_ref, y1s_ref, x2s_ref, y2s_ref, ids_ref, det_ref,
                out_ref):
    slot = (lax.broadcasted_iota(jnp.int32, (8, _C), 0) * _C
            + lax.broadcasted_iota(jnp.int32, (8, _C), 1))
    real = slot < _K
    x1 = jnp.where(real, x1_ref[...], 0.0)
    y1 = jnp.where(real, y1_ref[...], 0.0)
    x2 = jnp.where(real, x2_ref[...], 0.0)
    y2 = jnp.where(real, y2_ref[...], 0.0)
    sw0 = jnp.where(real, sc_ref[...], _NEG)
    area_b = jnp.maximum(x2 - x1, 0.0) * jnp.maximum(y2 - y1, 0.0)
    neg_half = _NEG / 2

    def nms_body(i, sw):
        m = jnp.max(sw)
        is_valid = m > neg_half
        sbest = jnp.min(jnp.where(sw == m, slot, _KPAD))
        bsel = slot == sbest
        bx1 = x1s_ref[pl.ds(sbest, 1), 0][0]
        by1 = y1s_ref[pl.ds(sbest, 1), 0][0]
        bx2 = x2s_ref[pl.ds(sbest, 1), 0][0]
        by2 = y2s_ref[pl.ds(sbest, 1), 0][0]
        ix1 = jnp.maximum(bx1, x1)
        iy1 = jnp.maximum(by1, y1)
        ix2 = jnp.minimum(bx2, x2)
        iy2 = jnp.minimum(by2, y2)
        inter = jnp.maximum(ix2 - ix1, 0.0) * jnp.maximum(iy2 - iy1, 0.0)
        area_a = jnp.maximum(bx2 - bx1, 0.0) * jnp.maximum(by2 - by1, 0.0)
        union = area_a + area_b - inter
        iou = inter / jnp.maximum(union, 1e-9)
        suppress = (iou > _NMS_THR) | bsel
        sw = jnp.where(is_valid & suppress, _NEG, sw)
        bi = jnp.minimum(ids_ref[pl.ds(sbest, 1), 0][0],
                         det_ref.shape[0] - 1)
        row = det_ref[pl.ds(bi, 1), :]
        out_ref[pl.ds(i, 1), :] = jnp.where(is_valid, row, 0.0)
        return sw

    lax.fori_loop(0, _MAX_OUT, nms_body, sw0)


def kernel(boxes, classification, detections):
    b = boxes[0]
    cls = classification[0]
    det = detections[0]
    n = b.shape[0]
    pad = _P - n
    cls_t = jnp.pad(cls.T, ((0, 0), (0, pad)),
                    constant_values=-1e30).reshape(cls.shape[1], _R, _C)

    scores, pos2 = pl.pallas_call(
        _score_kernel,
        out_shape=(
            jax.ShapeDtypeStruct((_R, _C), jnp.float32),
            jax.ShapeDtypeStruct((_R, _C), jnp.int32),
        ),
    )(cls_t)

    bp = jnp.pad(b, ((0, pad), (0, 0)))
    ar2 = jnp.arange(_P, dtype=jnp.int32)
    oidx, f0, f1, f2, f3, f4 = _sc_compact(
        pos2.reshape(_P), ar2, bp[:, 0], bp[:, 1], bp[:, 2], bp[:, 3],
        scores.reshape(_P))

    out = pl.pallas_call(
        _nms_kernel,
        out_shape=jax.ShapeDtypeStruct((_MAX_OUT, det.shape[1]), jnp.float32),
    )(f0.reshape(8, _C), f1.reshape(8, _C), f2.reshape(8, _C),
      f3.reshape(8, _C), f4.reshape(8, _C),
      f0[:, None], f1[:, None], f2[:, None], f3[:, None], oidx[:, None],
      det)
    return out[None]
